# Initial kernel scaffold; baseline (speedup 1.0000x reference)
import jax
import jax.numpy as jnp
from jax.experimental import pallas as pl


def kernel(x, real_table, imag_table):
    raise NotImplementedError("write your pallas kernel here")


# speedup vs baseline: 1.4679x; 1.4679x over previous
import functools

import jax
import jax.numpy as jnp
from jax import lax
from jax.experimental import pallas as pl
from jax.experimental.pallas import tpu as pltpu
from jax.experimental.pallas import tpu_sc as plsc

_VOCAB = 1000000
_FEAT = 32
_BATCH = 16384
_HIST = 20
_TOT = _BATCH * _HIST
_NW = 32
_PER_W = _TOT // _NW
_CHUNK = 128
_NCHUNK = _PER_W // _CHUNK

_mesh = plsc.VectorSubcoreMesh(core_axis_name="c", subcore_axis_name="s")


@functools.partial(
    pl.kernel,
    mesh=_mesh,
    out_type=[
        jax.ShapeDtypeStruct((_TOT, _FEAT), jnp.float32),
        jax.ShapeDtypeStruct((_TOT, _FEAT), jnp.float32),
    ],
    scratch_types=[
        pltpu.VMEM((_CHUNK,), jnp.int32),
        pltpu.VMEM((_CHUNK, _FEAT), jnp.float32),
        pltpu.VMEM((_CHUNK, _FEAT), jnp.float32),
        pltpu.SemaphoreType.DMA,
        pltpu.SemaphoreType.DMA,
    ],
    compiler_params=pltpu.CompilerParams(use_tc_tiling_on_sc=False),
)
def _embed_sc(x_hbm, rt_hbm, it_hbm, out_r, out_i,
              idx_v, rows_r, rows_i, sem_r, sem_i):
    wid = lax.axis_index("s") * 2 + lax.axis_index("c")
    base = wid * _PER_W

    def body(ci, carry):
        off = base + ci * _CHUNK
        pltpu.sync_copy(x_hbm.at[pl.ds(off, _CHUNK)], idx_v)
        cp_r = pltpu.async_copy(rt_hbm.at[idx_v], rows_r, sem_r)
        cp_i = pltpu.async_copy(it_hbm.at[idx_v], rows_i, sem_i)
        cp_r.wait()
        pltpu.sync_copy(rows_r, out_r.at[pl.ds(off, _CHUNK)])
        cp_i.wait()
        pltpu.sync_copy(rows_i, out_i.at[pl.ds(off, _CHUNK)])
        return carry

    lax.fori_loop(0, _NCHUNK, body, 0)


def kernel(x, real_table, imag_table):
    xf = x.reshape(-1).astype(jnp.int32)
    out_r, out_i = _embed_sc(xf, real_table, imag_table)
    return (out_r.reshape(_BATCH, _HIST, _FEAT),
            out_i.reshape(_BATCH, _HIST, _FEAT))


# speedup vs baseline: 1.5729x; 1.0716x over previous
import functools

import jax
import jax.numpy as jnp
from jax import lax
from jax.experimental import pallas as pl
from jax.experimental.pallas import tpu as pltpu
from jax.experimental.pallas import tpu_sc as plsc

_VOCAB = 1000000
_FEAT = 32
_BATCH = 16384
_HIST = 20
_TOT = _BATCH * _HIST
_NW = 32
_PER_W = _TOT // _NW
_CHUNK = 512
_NCHUNK = _PER_W // _CHUNK
_NBUF = 2

_mesh = plsc.VectorSubcoreMesh(core_axis_name="c", subcore_axis_name="s")


@functools.partial(
    pl.kernel,
    mesh=_mesh,
    out_type=[
        jax.ShapeDtypeStruct((_TOT, _FEAT), jnp.float32),
        jax.ShapeDtypeStruct((_TOT, _FEAT), jnp.float32),
    ],
    scratch_types=[
        [pltpu.VMEM((_CHUNK,), jnp.int32) for _ in range(_NBUF)],
        [pltpu.VMEM((_CHUNK, _FEAT), jnp.float32) for _ in range(_NBUF)],
        [pltpu.VMEM((_CHUNK, _FEAT), jnp.float32) for _ in range(_NBUF)],
        [pltpu.SemaphoreType.DMA for _ in range(_NBUF)],
        [pltpu.SemaphoreType.DMA for _ in range(_NBUF)],
        [pltpu.SemaphoreType.DMA for _ in range(_NBUF)],
        [pltpu.SemaphoreType.DMA for _ in range(_NBUF)],
    ],
    compiler_params=pltpu.CompilerParams(use_tc_tiling_on_sc=False),
)
def _embed_sc(x_hbm, rt_hbm, it_hbm, out_r, out_i,
              idx_v, rows_r, rows_i, gsem_r, gsem_i, wsem_r, wsem_i):
    wid = lax.axis_index("s") * 2 + lax.axis_index("c")
    base = wid * _PER_W

    def off(ci):
        return base + ci * _CHUNK

    pltpu.sync_copy(x_hbm.at[pl.ds(off(0), _CHUNK)], idx_v[0])
    g_r = [None] * _NBUF
    g_i = [None] * _NBUF
    w_r = [None] * _NBUF
    w_i = [None] * _NBUF
    g_r[0] = pltpu.async_copy(rt_hbm.at[idx_v[0]], rows_r[0], gsem_r[0])
    g_i[0] = pltpu.async_copy(it_hbm.at[idx_v[0]], rows_i[0], gsem_i[0])

    for ci in range(_NCHUNK):
        cur = ci % _NBUF
        nxt = (ci + 1) % _NBUF
        if ci + 1 < _NCHUNK:
            pltpu.sync_copy(x_hbm.at[pl.ds(off(ci + 1), _CHUNK)], idx_v[nxt])
            if w_r[nxt] is not None:
                w_r[nxt].wait()
                w_i[nxt].wait()
            g_r[nxt] = pltpu.async_copy(rt_hbm.at[idx_v[nxt]], rows_r[nxt],
                                        gsem_r[nxt])
            g_i[nxt] = pltpu.async_copy(it_hbm.at[idx_v[nxt]], rows_i[nxt],
                                        gsem_i[nxt])
        g_r[cur].wait()
        w_r[cur] = pltpu.async_copy(rows_r[cur], out_r.at[pl.ds(off(ci), _CHUNK)],
                                    wsem_r[cur])
        g_i[cur].wait()
        w_i[cur] = pltpu.async_copy(rows_i[cur], out_i.at[pl.ds(off(ci), _CHUNK)],
                                    wsem_i[cur])

    for b in range(_NBUF):
        w_r[b].wait()
        w_i[b].wait()


def kernel(x, real_table, imag_table):
    xf = x.reshape(-1).astype(jnp.int32)
    out_r, out_i = _embed_sc(xf, real_table, imag_table)
    return (out_r.reshape(_BATCH, _HIST, _FEAT),
            out_i.reshape(_BATCH, _HIST, _FEAT))
